# baseline (speedup 1.0000x reference)
import functools

import jax
import jax.numpy as jnp
from jax.experimental import pallas as pl
from jax.experimental.pallas import tpu as pltpu


def _gcn_block(scal_ref, *refs):
    *adj_refs, x_ref, h0_ref, w_ref, out_ref = refs
    s = scal_ref[0, 0]
    theta = scal_ref[0, 1]
    x = x_ref[...].astype(jnp.bfloat16)
    w = w_ref[...].astype(jnp.bfloat16)
    chunk = adj_refs[0].shape[0]
    for idx, a_ref in enumerate(adj_refs):
        a = a_ref[...].astype(jnp.bfloat16)
        hi = jax.lax.dot_general(
            a, x, (((1,), (0,)), ((), ())), preferred_element_type=jnp.float32
        )
        rows = pl.ds(idx * chunk, chunk)
        support = (1.0 - s) * hi + s * h0_ref[rows, :]
        sw = jax.lax.dot_general(
            support.astype(jnp.bfloat16),
            w,
            (((1,), (0,)), ((), ())),
            preferred_element_type=jnp.float32,
        )
        out_ref[rows, :] = theta * sw + (1.0 - theta) * support


@functools.partial(jax.jit, static_argnames=("block_m", "streams"))
def _gcn(input, adj, h0, weight, s, theta, block_m=400, streams=2):
    n, d_in = input.shape
    d_out = weight.shape[1]
    chunk = block_m // streams
    scal = jnp.reshape(jnp.stack([s, theta]).astype(jnp.float32), (1, 2))
    adj_specs = [
        pl.BlockSpec((chunk, n), functools.partial(lambda j, i: (streams * i + j, 0), j))
        for j in range(streams)
    ]
    return pl.pallas_call(
        _gcn_block,
        grid=(n // block_m,),
        in_specs=[
            pl.BlockSpec(memory_space=pltpu.SMEM),
            *adj_specs,
            pl.BlockSpec((n, d_in), lambda i: (0, 0)),
            pl.BlockSpec((block_m, d_in), lambda i: (i, 0)),
            pl.BlockSpec((d_in, d_out), lambda i: (0, 0)),
        ],
        out_specs=pl.BlockSpec((block_m, d_out), lambda i: (i, 0)),
        out_shape=jax.ShapeDtypeStruct((n, d_out), jnp.float32),
        compiler_params=pltpu.CompilerParams(
            dimension_semantics=("parallel",),
        ),
    )(scal, *([adj] * streams), input, h0, weight)


def kernel(input, adj, h0, weight, lamda, s, l):
    theta = (lamda / l).astype(jnp.float32)
    s = jnp.asarray(s, jnp.float32)
    return _gcn(input, adj, h0, weight, s, theta, block_m=400, streams=5)


# speedup vs baseline: 1.0310x; 1.0310x over previous
import functools

import jax
import jax.numpy as jnp
from jax.experimental import pallas as pl
from jax.experimental.pallas import tpu as pltpu


def _gcn_block(scal_ref, *refs):
    *adj_refs, x_ref, h0_ref, w_ref, out_ref = refs
    s = scal_ref[0, 0]
    theta = scal_ref[0, 1]
    x = x_ref[...].astype(jnp.bfloat16)
    w = w_ref[...].astype(jnp.bfloat16)
    chunk = adj_refs[0].shape[0]
    for idx, a_ref in enumerate(adj_refs):
        a = a_ref[...].astype(jnp.bfloat16)
        hi = jax.lax.dot_general(
            a, x, (((1,), (0,)), ((), ())), preferred_element_type=jnp.float32
        )
        rows = pl.ds(idx * chunk, chunk)
        support = (1.0 - s) * hi + s * h0_ref[rows, :]
        sw = jax.lax.dot_general(
            support.astype(jnp.bfloat16),
            w,
            (((1,), (0,)), ((), ())),
            preferred_element_type=jnp.float32,
        )
        out_ref[rows, :] = theta * sw + (1.0 - theta) * support


@functools.partial(jax.jit, static_argnames=("block_m", "streams"))
def _gcn(input, adj, h0, weight, s, theta, block_m=400, streams=2):
    n, d_in = input.shape
    d_out = weight.shape[1]
    chunk = block_m // streams
    scal = jnp.reshape(jnp.stack([s, theta]).astype(jnp.float32), (1, 2))
    adj_specs = [
        pl.BlockSpec((chunk, n), functools.partial(lambda j, i: (streams * i + j, 0), j))
        for j in range(streams)
    ]
    return pl.pallas_call(
        _gcn_block,
        grid=(n // block_m,),
        in_specs=[
            pl.BlockSpec(memory_space=pltpu.SMEM),
            *adj_specs,
            pl.BlockSpec((n, d_in), lambda i: (0, 0)),
            pl.BlockSpec((block_m, d_in), lambda i: (i, 0)),
            pl.BlockSpec((d_in, d_out), lambda i: (0, 0)),
        ],
        out_specs=pl.BlockSpec((block_m, d_out), lambda i: (i, 0)),
        out_shape=jax.ShapeDtypeStruct((n, d_out), jnp.float32),
        compiler_params=pltpu.CompilerParams(
            dimension_semantics=("parallel",),
        ),
    )(scal, *([adj] * streams), input, h0, weight)


def kernel(input, adj, h0, weight, lamda, s, l):
    theta = (lamda / l).astype(jnp.float32)
    s = jnp.asarray(s, jnp.float32)
    return _gcn(input, adj, h0, weight, s, theta, block_m=400, streams=2)
